# Initial kernel scaffold; baseline (speedup 1.0000x reference)
import jax
import jax.numpy as jnp
from jax.experimental import pallas as pl


def kernel(u_feature, i_feature, u2i_edge_index, u2i_edge_weight, u2e_edge_index, u2e_edge_weight, i2e_edge_index, i2e_edge_weight, u2e_Wt, u2e_bt, u2e_Wi, u2e_bi, i2e_Wt, i2e_bt, i2e_Wi, i2e_bi, u2i_Wt, u2i_bt, u2i_Wi, u2i_bi, uatt_W1, uatt_b1, uatt_W2, iatt_W1, iatt_b1, iatt_W2):
    raise NotImplementedError("write your pallas kernel here")


# speedup vs baseline: 3.2547x; 3.2547x over previous
import functools

import jax
import jax.numpy as jnp
from jax import lax
from jax.experimental import pallas as pl
from jax.experimental.pallas import tpu as pltpu
from jax.experimental.pallas import tpu_sc as plsc

N_U = 10000
N_I = 10000
FDIM = 128
NS = 16
NC = 2
EB = 128


def _dense_cell(x, Wt, bt, Wi, bi):
    n = x.shape[0]
    bn = 2000
    nb = n // bn

    def body(x_ref, wt_ref, bt_ref, wi_ref, bi_ref, g_ref):
        h = pl.program_id(1)
        xv = x_ref[...]
        h1 = jnp.dot(xv, wt_ref[...], preferred_element_type=jnp.float32) + bt_ref[...]
        h12 = h1 + jnp.dot(xv * xv, wi_ref[...], preferred_element_type=jnp.float32) + bi_ref[...]
        g_ref[...] = jnp.where(h == 0, h12[:, :64], h12[:, 64:])

    return pl.pallas_call(
        body,
        grid=(nb, 2),
        in_specs=[
            pl.BlockSpec((bn, FDIM), lambda i, h: (i, 0)),
            pl.BlockSpec((FDIM, FDIM), lambda i, h: (0, 0)),
            pl.BlockSpec((1, FDIM), lambda i, h: (0, 0)),
            pl.BlockSpec((FDIM, FDIM), lambda i, h: (0, 0)),
            pl.BlockSpec((1, FDIM), lambda i, h: (0, 0)),
        ],
        out_specs=pl.BlockSpec((bn, 64), lambda i, h: (h * nb + i, 0)),
        out_shape=jax.ShapeDtypeStruct((2 * n, 64), jnp.float32),
    )(x, Wt, bt.reshape(1, FDIM), Wi, bi.reshape(1, FDIM))


@functools.lru_cache(maxsize=None)
def _make_spmm(n, e_pad):
    n_chunks = e_pad // NS // EB
    e_per_tile = e_pad // NS
    dump_rows = 200
    dump_chunks = n // dump_rows
    dump_iters = (dump_chunks + NS - 1) // NS
    mesh = plsc.VectorSubcoreMesh(
        core_axis_name="c", subcore_axis_name="s", num_cores=NC, num_subcores=NS)

    @functools.partial(
        pl.kernel,
        out_type=jax.ShapeDtypeStruct((2 * n, 64), jnp.float32),
        mesh=mesh,
        scratch_types=[
            pltpu.VMEM((EB,), jnp.int32),
            pltpu.VMEM((EB,), jnp.int32),
            pltpu.VMEM((EB,), jnp.float32),
            pltpu.VMEM((EB, 64), jnp.float32),
            pltpu.VMEM((dump_rows, 64), jnp.float32),
            pltpu.VMEM_SHARED((n, 64), jnp.float32),
            pltpu.SemaphoreType.DMA,
        ],
        compiler_params=pltpu.CompilerParams(use_tc_tiling_on_sc=False),
    )
    def spmm(g_hbm, src_hbm, dst_hbm, w_hbm, out_hbm,
             src_v, dst_v, w_v, rows_v, zbuf, acc, sem):
        c = lax.axis_index("c")
        s = lax.axis_index("s")
        cn = c * n

        def zb(j, carry):
            for k in range(4):
                zbuf[j, pl.ds(16 * k, 16)] = jnp.zeros((16,), jnp.float32)
            return carry
        lax.fori_loop(0, dump_rows, zb, 0)
        for t in range(dump_iters):
            q = s + NS * t
            @pl.when(q < dump_chunks)
            def _():
                pltpu.sync_copy(zbuf, acc.at[pl.ds(q * dump_rows, dump_rows)])
        plsc.subcore_barrier()

        base = s * e_per_tile

        def chunk(t, carry):
            off = base + t * EB
            pltpu.sync_copy(src_hbm.at[pl.ds(off, EB)], src_v)
            pltpu.sync_copy(dst_hbm.at[pl.ds(off, EB)], dst_v)
            pltpu.sync_copy(w_hbm.at[pl.ds(off, EB)], w_v)
            for k in range(EB // 16):
                sl = pl.ds(16 * k, 16)
                src_v[sl] = src_v[sl] + cn
            pltpu.async_copy(g_hbm.at[src_v], rows_v, sem).wait()

            def sedge(g, cc):
                wvec = w_v[pl.ds(16 * g, 16)]
                for jj in range(16):
                    wj = wvec[jj]
                    j = 16 * g + jj
                    for k in range(4):
                        sl = pl.ds(16 * k, 16)
                        rows_v[j, sl] = rows_v[j, sl] * wj
                return cc
            lax.fori_loop(0, EB // 16, sedge, 0)

            pltpu.sync_copy(rows_v, acc.at[dst_v], add=True)
            return carry

        lax.fori_loop(0, n_chunks, chunk, 0)
        plsc.subcore_barrier()

        for t in range(dump_iters):
            q = s + NS * t
            @pl.when(q < dump_chunks)
            def _():
                lo = q * dump_rows
                pltpu.sync_copy(acc.at[pl.ds(lo, dump_rows)],
                                out_hbm.at[pl.ds(cn + lo, dump_rows)])

    return spmm


def _prep_edges(edge_index, edge_weight, e_pad, n):
    e = edge_index.shape[1]
    pad = e_pad - e
    pad_idx = jnp.arange(pad, dtype=jnp.int32) % n
    src = jnp.concatenate([edge_index[1], pad_idx])
    dst = jnp.concatenate([edge_index[0], pad_idx])
    w = jnp.concatenate([edge_weight, jnp.zeros((pad,), jnp.float32)])
    return src, dst, w


def _attention(S_e, b0e, b1e, S_ui, b0ui, b1ui, x, Wte, bte, Wtui, btui, W1, b1, W2):
    n = x.shape[0]
    bn = 2000
    nb = n // bn

    def body(se0, se1, su0, su1, x_ref, wte, bter, wtui, btuir, w1, b1r, w2, out_ref):
        xv = x_ref[...]
        z0 = (jnp.concatenate([se0[...], se1[...]], axis=1)
              + jnp.dot(xv, wte[...], preferred_element_type=jnp.float32) + bter[...])
        z1 = (jnp.concatenate([su0[...], su1[...]], axis=1)
              + jnp.dot(xv, wtui[...], preferred_element_type=jnp.float32) + btuir[...])
        t0 = jnp.tanh(jnp.dot(z0, w1[...], preferred_element_type=jnp.float32) + b1r[...])
        t1 = jnp.tanh(jnp.dot(z1, w1[...], preferred_element_type=jnp.float32) + b1r[...])
        s0 = jnp.sum(t0 * w2[...], axis=1, keepdims=True)
        s1 = jnp.sum(t1 * w2[...], axis=1, keepdims=True)
        m = jnp.maximum(s0, s1)
        e0 = jnp.exp(s0 - m)
        e1 = jnp.exp(s1 - m)
        out_ref[...] = (e0 * z0 + e1 * z1) / (e0 + e1)

    return pl.pallas_call(
        body,
        grid=(nb,),
        in_specs=[
            pl.BlockSpec((bn, 64), lambda i, b=b0e // bn: (b + i, 0)),
            pl.BlockSpec((bn, 64), lambda i, b=b1e // bn: (b + i, 0)),
            pl.BlockSpec((bn, 64), lambda i, b=b0ui // bn: (b + i, 0)),
            pl.BlockSpec((bn, 64), lambda i, b=b1ui // bn: (b + i, 0)),
            pl.BlockSpec((bn, FDIM), lambda i: (i, 0)),
            pl.BlockSpec((FDIM, FDIM), lambda i: (0, 0)),
            pl.BlockSpec((1, FDIM), lambda i: (0, 0)),
            pl.BlockSpec((FDIM, FDIM), lambda i: (0, 0)),
            pl.BlockSpec((1, FDIM), lambda i: (0, 0)),
            pl.BlockSpec((FDIM, 32), lambda i: (0, 0)),
            pl.BlockSpec((1, 32), lambda i: (0, 0)),
            pl.BlockSpec((1, 32), lambda i: (0, 0)),
        ],
        out_specs=pl.BlockSpec((bn, FDIM), lambda i: (i, 0)),
        out_shape=jax.ShapeDtypeStruct((n, FDIM), jnp.float32),
    )(S_e, S_e, S_ui, S_ui, x, Wte, bte.reshape(1, FDIM), Wtui,
      btui.reshape(1, FDIM), W1, b1.reshape(1, 32), W2.reshape(1, 32))


def kernel(u_feature, i_feature, u2i_edge_index, u2i_edge_weight,
           u2e_edge_index, u2e_edge_weight, i2e_edge_index, i2e_edge_weight,
           u2e_Wt, u2e_bt, u2e_Wi, u2e_bi,
           i2e_Wt, i2e_bt, i2e_Wi, i2e_bi,
           u2i_Wt, u2i_bt, u2i_Wi, u2i_bi,
           uatt_W1, uatt_b1, uatt_W2,
           iatt_W1, iatt_b1, iatt_W2):
    n_ui = N_U + N_I
    feats = jnp.concatenate([u_feature, i_feature], axis=0)

    g_e = _dense_cell(u_feature, u2e_Wt, u2e_bt, u2e_Wi, u2e_bi)
    g_i = _dense_cell(i_feature, i2e_Wt, i2e_bt, i2e_Wi, i2e_bi)
    g_ui = _dense_cell(feats, u2i_Wt, u2i_bt, u2i_Wi, u2i_bi)

    e_small = ((160000 + NS * EB - 1) // (NS * EB)) * NS * EB
    e_big = ((320000 + NS * EB - 1) // (NS * EB)) * NS * EB
    se_src, se_dst, se_w = _prep_edges(u2e_edge_index, u2e_edge_weight, e_small, N_U)
    si_src, si_dst, si_w = _prep_edges(i2e_edge_index, i2e_edge_weight, e_small, N_I)
    sui_src, sui_dst, sui_w = _prep_edges(u2i_edge_index, u2i_edge_weight, e_big, n_ui)

    spmm_small = _make_spmm(N_U, e_small)
    spmm_big = _make_spmm(n_ui, e_big)
    s_e = spmm_small(g_e, se_src, se_dst, se_w)
    s_i = spmm_small(g_i, si_src, si_dst, si_w)
    s_ui = spmm_big(g_ui, sui_src, sui_dst, sui_w)

    u_out = _attention(s_e, 0, N_U, s_ui, 0, n_ui, u_feature,
                       u2e_Wt, u2e_bt, u2i_Wt, u2i_bt, uatt_W1, uatt_b1, uatt_W2)
    i_out = _attention(s_i, 0, N_I, s_ui, N_U, n_ui + N_U, i_feature,
                       i2e_Wt, i2e_bt, u2i_Wt, u2i_bt, iatt_W1, iatt_b1, iatt_W2)
    return (u_out, i_out)


# speedup vs baseline: 5.4782x; 1.6832x over previous
import functools

import jax
import jax.numpy as jnp
from jax import lax
from jax.experimental import pallas as pl
from jax.experimental.pallas import tpu as pltpu
from jax.experimental.pallas import tpu_sc as plsc

N_U = 10000
N_I = 10000
FDIM = 128
NS = 16
NC = 2
EB = 128


def _dense_cell(x, Wt, bt, Wi, bi):
    n = x.shape[0]
    bn = 2000
    nb = n // bn

    def body(x_ref, wt_ref, bt_ref, wi_ref, bi_ref, g_ref):
        h = pl.program_id(1)
        xv = x_ref[...]
        h1 = jnp.dot(xv, wt_ref[...], preferred_element_type=jnp.float32) + bt_ref[...]
        h12 = h1 + jnp.dot(xv * xv, wi_ref[...], preferred_element_type=jnp.float32) + bi_ref[...]
        g_ref[...] = jnp.where(h == 0, h12[:, :64], h12[:, 64:])

    return pl.pallas_call(
        body,
        grid=(nb, 2),
        in_specs=[
            pl.BlockSpec((bn, FDIM), lambda i, h: (i, 0)),
            pl.BlockSpec((FDIM, FDIM), lambda i, h: (0, 0)),
            pl.BlockSpec((1, FDIM), lambda i, h: (0, 0)),
            pl.BlockSpec((FDIM, FDIM), lambda i, h: (0, 0)),
            pl.BlockSpec((1, FDIM), lambda i, h: (0, 0)),
        ],
        out_specs=pl.BlockSpec((bn, 64), lambda i, h: (h * nb + i, 0)),
        out_shape=jax.ShapeDtypeStruct((2 * n, 64), jnp.float32),
    )(x, Wt, bt.reshape(1, FDIM), Wi, bi.reshape(1, FDIM))


@functools.lru_cache(maxsize=None)
def _make_spmm(n, n_chunks):
    e_per_tile = n_chunks * EB
    dump_rows = 200
    dump_chunks = n // dump_rows
    dump_iters = (dump_chunks + NS - 1) // NS
    mesh = plsc.VectorSubcoreMesh(
        core_axis_name="c", subcore_axis_name="s", num_cores=NC, num_subcores=NS)

    @functools.partial(
        pl.kernel,
        out_type=jax.ShapeDtypeStruct((2 * n, 64), jnp.float32),
        mesh=mesh,
        scratch_types=[
            pltpu.VMEM((3, EB), jnp.int32),
            pltpu.VMEM((3, EB), jnp.int32),
            pltpu.VMEM((EB,), jnp.int32),
            pltpu.VMEM((EB,), jnp.int32),
            pltpu.VMEM((EB, 64), jnp.float32),
            pltpu.VMEM((EB, 64), jnp.float32),
            pltpu.VMEM((dump_rows, 64), jnp.float32),
            pltpu.VMEM_SHARED((n, 64), jnp.float32),
            pltpu.SemaphoreType.DMA,
            pltpu.SemaphoreType.DMA,
            pltpu.SemaphoreType.DMA,
            pltpu.SemaphoreType.DMA,
            pltpu.SemaphoreType.DMA,
            pltpu.SemaphoreType.DMA,
        ],
        compiler_params=pltpu.CompilerParams(
            use_tc_tiling_on_sc=False, needs_layout_passes=False),
    )
    def spmm(g_hbm, edata_hbm, out_hbm,
             ibuf0, ibuf1, dbuf0, dbuf1, rows0, rows1, zbuf, acc,
             si0, si1, sg0, sg1, sc0, sc1):
        c = lax.axis_index("c")
        s = lax.axis_index("s")
        cn = c * n
        ibuf = (ibuf0, ibuf1)
        dbuf = (dbuf0, dbuf1)
        rows = (rows0, rows1)
        sem_i = (si0, si1)
        sem_g = (sg0, sg1)
        sem_c = (sc0, sc1)
        base = s * e_per_tile

        def idx_desc(t, b):
            tc = jnp.minimum(t, n_chunks - 1)
            return pltpu.make_async_copy(
                edata_hbm.at[:, pl.ds(base + tc * EB, EB)], ibuf[b], sem_i[b])

        def gather_desc(t, b):
            return pltpu.make_async_copy(
                g_hbm.at[ibuf[b].at[0]], rows[b], sem_g[b])

        def scatter_start(b):
            pltpu.async_copy(rows[b], acc.at[dbuf[b]], sem_c[b], add=True)

        def scatter_wait(b):
            pltpu.make_async_copy(rows[b], acc.at[dbuf[b]], sem_c[b]).wait()

        def adjust(b):
            for k in range(EB // 16):
                sl = pl.ds(16 * k, 16)
                ibuf[b][0, sl] = ibuf[b][0, sl] + cn

        def scale(b):
            def sedge(g, cc):
                wvec = plsc.bitcast(ibuf[b][2, pl.ds(16 * g, 16)], jnp.float32)
                for jj in range(16):
                    wj = wvec[jj]
                    j = 16 * g + jj
                    for k in range(4):
                        sl = pl.ds(16 * k, 16)
                        rows[b][j, sl] = rows[b][j, sl] * wj
                return cc
            lax.fori_loop(0, EB // 16, sedge, 0)

        def zb(j, carry):
            for k in range(4):
                zbuf[j, pl.ds(16 * k, 16)] = jnp.zeros((16,), jnp.float32)
            return carry
        lax.fori_loop(0, dump_rows, zb, 0)
        for t in range(dump_iters):
            q = s + NS * t
            @pl.when(q < dump_chunks)
            def _():
                pltpu.sync_copy(zbuf, acc.at[pl.ds(q * dump_rows, dump_rows)])
        plsc.subcore_barrier()

        idx_desc(0, 0).start()
        idx_desc(1, 1).start()
        idx_desc(0, 0).wait()
        adjust(0)
        gather_desc(0, 0).start()

        def half(t, b):
            o = 1 - b
            idx_desc(t + 1, o).wait()
            adjust(o)
            @pl.when(t > 0)
            def _():
                scatter_wait(o)
            gather_desc(t + 1, o).start()
            gather_desc(t, b).wait()
            scale(b)
            for k in range(EB // 16):
                sl = pl.ds(16 * k, 16)
                dbuf[b][sl] = ibuf[b][1, sl]
            scatter_start(b)
            idx_desc(t + 2, b).start()

        def body(tt, carry):
            half(2 * tt, 0)
            half(2 * tt + 1, 1)
            return carry
        lax.fori_loop(0, n_chunks // 2, body, 0)

        idx_desc(n_chunks + 1, 1).wait()
        gather_desc(n_chunks, 0).wait()
        scatter_wait(1)
        plsc.subcore_barrier()

        for t in range(dump_iters):
            q = s + NS * t
            @pl.when(q < dump_chunks)
            def _():
                lo = q * dump_rows
                pltpu.sync_copy(acc.at[pl.ds(lo, dump_rows)],
                                out_hbm.at[pl.ds(cn + lo, dump_rows)])

    return spmm


def _prep_edges(edge_index, edge_weight, e_pad, n):
    e = edge_index.shape[1]
    pad = e_pad - e
    pad_idx = jnp.arange(pad, dtype=jnp.int32) % n
    src = jnp.concatenate([edge_index[1], pad_idx])
    dst = jnp.concatenate([edge_index[0], pad_idx])
    w = jnp.concatenate([edge_weight, jnp.zeros((pad,), jnp.float32)])
    return jnp.stack([src, dst, jax.lax.bitcast_convert_type(w, jnp.int32)])


def _attention(S_e, b0e, b1e, S_ui, b0ui, b1ui, x, Wte, bte, Wtui, btui, W1, b1, W2):
    n = x.shape[0]
    bn = 2000
    nb = n // bn

    def body(se0, se1, su0, su1, x_ref, wte, bter, wtui, btuir, w1, b1r, w2, out_ref):
        xv = x_ref[...]
        z0 = (jnp.concatenate([se0[...], se1[...]], axis=1)
              + jnp.dot(xv, wte[...], preferred_element_type=jnp.float32) + bter[...])
        z1 = (jnp.concatenate([su0[...], su1[...]], axis=1)
              + jnp.dot(xv, wtui[...], preferred_element_type=jnp.float32) + btuir[...])
        t0 = jnp.tanh(jnp.dot(z0, w1[...], preferred_element_type=jnp.float32) + b1r[...])
        t1 = jnp.tanh(jnp.dot(z1, w1[...], preferred_element_type=jnp.float32) + b1r[...])
        s0 = jnp.sum(t0 * w2[...], axis=1, keepdims=True)
        s1 = jnp.sum(t1 * w2[...], axis=1, keepdims=True)
        m = jnp.maximum(s0, s1)
        e0 = jnp.exp(s0 - m)
        e1 = jnp.exp(s1 - m)
        out_ref[...] = (e0 * z0 + e1 * z1) / (e0 + e1)

    return pl.pallas_call(
        body,
        grid=(nb,),
        in_specs=[
            pl.BlockSpec((bn, 64), lambda i, b=b0e // bn: (b + i, 0)),
            pl.BlockSpec((bn, 64), lambda i, b=b1e // bn: (b + i, 0)),
            pl.BlockSpec((bn, 64), lambda i, b=b0ui // bn: (b + i, 0)),
            pl.BlockSpec((bn, 64), lambda i, b=b1ui // bn: (b + i, 0)),
            pl.BlockSpec((bn, FDIM), lambda i: (i, 0)),
            pl.BlockSpec((FDIM, FDIM), lambda i: (0, 0)),
            pl.BlockSpec((1, FDIM), lambda i: (0, 0)),
            pl.BlockSpec((FDIM, FDIM), lambda i: (0, 0)),
            pl.BlockSpec((1, FDIM), lambda i: (0, 0)),
            pl.BlockSpec((FDIM, 32), lambda i: (0, 0)),
            pl.BlockSpec((1, 32), lambda i: (0, 0)),
            pl.BlockSpec((1, 32), lambda i: (0, 0)),
        ],
        out_specs=pl.BlockSpec((bn, FDIM), lambda i: (i, 0)),
        out_shape=jax.ShapeDtypeStruct((n, FDIM), jnp.float32),
    )(S_e, S_e, S_ui, S_ui, x, Wte, bte.reshape(1, FDIM), Wtui,
      btui.reshape(1, FDIM), W1, b1.reshape(1, 32), W2.reshape(1, 32))


def kernel(u_feature, i_feature, u2i_edge_index, u2i_edge_weight,
           u2e_edge_index, u2e_edge_weight, i2e_edge_index, i2e_edge_weight,
           u2e_Wt, u2e_bt, u2e_Wi, u2e_bi,
           i2e_Wt, i2e_bt, i2e_Wi, i2e_bi,
           u2i_Wt, u2i_bt, u2i_Wi, u2i_bi,
           uatt_W1, uatt_b1, uatt_W2,
           iatt_W1, iatt_b1, iatt_W2):
    n_ui = N_U + N_I
    feats = jnp.concatenate([u_feature, i_feature], axis=0)

    g_e = _dense_cell(u_feature, u2e_Wt, u2e_bt, u2e_Wi, u2e_bi)
    g_i = _dense_cell(i_feature, i2e_Wt, i2e_bt, i2e_Wi, i2e_bi)
    g_ui = _dense_cell(feats, u2i_Wt, u2i_bt, u2i_Wi, u2i_bi)

    nch_small = -(-160000 // (NS * EB * 2)) * 2
    nch_big = -(-320000 // (NS * EB * 2)) * 2
    e_small = NS * nch_small * EB
    e_big = NS * nch_big * EB
    ed_e = _prep_edges(u2e_edge_index, u2e_edge_weight, e_small, N_U)
    ed_i = _prep_edges(i2e_edge_index, i2e_edge_weight, e_small, N_I)
    ed_ui = _prep_edges(u2i_edge_index, u2i_edge_weight, e_big, n_ui)

    spmm_small = _make_spmm(N_U, nch_small)
    spmm_big = _make_spmm(n_ui, nch_big)
    s_e = spmm_small(g_e, ed_e)
    s_i = spmm_small(g_i, ed_i)
    s_ui = spmm_big(g_ui, ed_ui)

    u_out = _attention(s_e, 0, N_U, s_ui, 0, n_ui, u_feature,
                       u2e_Wt, u2e_bt, u2i_Wt, u2i_bt, uatt_W1, uatt_b1, uatt_W2)
    i_out = _attention(s_i, 0, N_I, s_ui, N_U, n_ui + N_U, i_feature,
                       i2e_Wt, i2e_bt, u2i_Wt, u2i_bt, iatt_W1, iatt_b1, iatt_W2)
    return (u_out, i_out)


# speedup vs baseline: 11.7369x; 2.1425x over previous
import functools

import jax
import jax.numpy as jnp
from jax import lax
from jax.experimental import pallas as pl
from jax.experimental.pallas import tpu as pltpu
from jax.experimental.pallas import tpu_sc as plsc

N_U = 10000
N_I = 10000
FDIM = 128
NS = 16
NC = 2
EB = 128


def _dense_cell(x, Wt, bt, Wi, bi):
    n = x.shape[0]
    bn = 2000
    nb = n // bn

    def body(x_ref, wt_ref, bt_ref, wi_ref, bi_ref, g_ref):
        h = pl.program_id(1)
        xv = x_ref[...]
        h1 = jnp.dot(xv, wt_ref[...], preferred_element_type=jnp.float32) + bt_ref[...]
        h12 = h1 + jnp.dot(xv * xv, wi_ref[...], preferred_element_type=jnp.float32) + bi_ref[...]
        g_ref[...] = jnp.where(h == 0, h12[:, :64], h12[:, 64:])

    return pl.pallas_call(
        body,
        grid=(nb, 2),
        in_specs=[
            pl.BlockSpec((bn, FDIM), lambda i, h: (i, 0)),
            pl.BlockSpec((FDIM, FDIM), lambda i, h: (0, 0)),
            pl.BlockSpec((1, FDIM), lambda i, h: (0, 0)),
            pl.BlockSpec((FDIM, FDIM), lambda i, h: (0, 0)),
            pl.BlockSpec((1, FDIM), lambda i, h: (0, 0)),
        ],
        out_specs=pl.BlockSpec((bn, 64), lambda i, h: (h * nb + i, 0)),
        out_shape=jax.ShapeDtypeStruct((2 * n, 64), jnp.float32),
    )(x, Wt, bt.reshape(1, FDIM), Wi, bi.reshape(1, FDIM))


@functools.lru_cache(maxsize=None)
def _make_spmm(n, n_chunks):
    e_per_tile = n_chunks * EB
    dump_rows = 200
    dump_chunks = n // dump_rows
    dump_iters = (dump_chunks + NS - 1) // NS
    mesh = plsc.VectorSubcoreMesh(
        core_axis_name="c", subcore_axis_name="s", num_cores=NC, num_subcores=NS)

    @functools.partial(
        pl.kernel,
        out_type=jax.ShapeDtypeStruct((2 * n, 64), jnp.float32),
        mesh=mesh,
        scratch_types=[
            pltpu.VMEM((3, EB), jnp.int32),
            pltpu.VMEM((3, EB), jnp.int32),
            pltpu.VMEM((EB,), jnp.int32),
            pltpu.VMEM((EB,), jnp.int32),
            pltpu.VMEM((EB, 64), jnp.float32),
            pltpu.VMEM((EB, 64), jnp.float32),
            pltpu.VMEM((dump_rows, 64), jnp.float32),
            pltpu.VMEM_SHARED((n, 64), jnp.float32),
            pltpu.SemaphoreType.DMA,
            pltpu.SemaphoreType.DMA,
            pltpu.SemaphoreType.DMA,
            pltpu.SemaphoreType.DMA,
            pltpu.SemaphoreType.DMA,
            pltpu.SemaphoreType.DMA,
        ],
        compiler_params=pltpu.CompilerParams(
            use_tc_tiling_on_sc=False, needs_layout_passes=False),
    )
    def spmm(g_hbm, edata_hbm, out_hbm,
             ibuf0, ibuf1, dbuf0, dbuf1, rows0, rows1, zbuf, acc,
             si0, si1, sg0, sg1, sc0, sc1):
        c = lax.axis_index("c")
        s = lax.axis_index("s")
        cn = c * n
        ibuf = (ibuf0, ibuf1)
        dbuf = (dbuf0, dbuf1)
        rows = (rows0, rows1)
        sem_i = (si0, si1)
        sem_g = (sg0, sg1)
        sem_c = (sc0, sc1)
        base = s * e_per_tile

        def idx_desc(t, b):
            tc = jnp.minimum(t, n_chunks - 1)
            return pltpu.make_async_copy(
                edata_hbm.at[:, pl.ds(base + tc * EB, EB)], ibuf[b], sem_i[b])

        def gather_desc(t, b):
            return pltpu.make_async_copy(
                g_hbm.at[ibuf[b].at[0]], rows[b], sem_g[b])

        def scatter_start(b):
            pltpu.async_copy(rows[b], acc.at[dbuf[b]], sem_c[b], add=True)

        def scatter_wait(b):
            pltpu.make_async_copy(rows[b], acc.at[dbuf[b]], sem_c[b]).wait()

        def adjust(b):
            for k in range(EB // 16):
                sl = pl.ds(16 * k, 16)
                ibuf[b][0, sl] = ibuf[b][0, sl] + cn

        def scale(b):
            @plsc.parallel_loop(0, EB // 16, unroll=2)
            def _(g):
                wvec = plsc.bitcast(ibuf[b][2, pl.ds(16 * g, 16)], jnp.float32)
                for jj in range(16):
                    wj = wvec[jj]
                    j = 16 * g + jj
                    for k in range(4):
                        sl = pl.ds(16 * k, 16)
                        rows[b][j, sl] = rows[b][j, sl] * wj

        def zb(j, carry):
            for k in range(4):
                zbuf[j, pl.ds(16 * k, 16)] = jnp.zeros((16,), jnp.float32)
            return carry
        lax.fori_loop(0, dump_rows, zb, 0)
        for t in range(dump_iters):
            q = s + NS * t
            @pl.when(q < dump_chunks)
            def _():
                pltpu.sync_copy(zbuf, acc.at[pl.ds(q * dump_rows, dump_rows)])
        plsc.subcore_barrier()

        idx_desc(0, 0).start()
        idx_desc(1, 1).start()
        idx_desc(0, 0).wait()
        adjust(0)
        gather_desc(0, 0).start()

        def half(t, b):
            o = 1 - b
            idx_desc(t + 1, o).wait()
            adjust(o)
            @pl.when(t > 0)
            def _():
                scatter_wait(o)
            gather_desc(t + 1, o).start()
            gather_desc(t, b).wait()
            scale(b)
            for k in range(EB // 16):
                sl = pl.ds(16 * k, 16)
                dbuf[b][sl] = ibuf[b][1, sl]
            scatter_start(b)
            idx_desc(t + 2, b).start()

        def body(tt, carry):
            half(2 * tt, 0)
            half(2 * tt + 1, 1)
            return carry
        lax.fori_loop(0, n_chunks // 2, body, 0)

        idx_desc(n_chunks + 1, 1).wait()
        gather_desc(n_chunks, 0).wait()
        scatter_wait(1)
        plsc.subcore_barrier()

        for t in range(dump_iters):
            q = s + NS * t
            @pl.when(q < dump_chunks)
            def _():
                lo = q * dump_rows
                pltpu.sync_copy(acc.at[pl.ds(lo, dump_rows)],
                                out_hbm.at[pl.ds(cn + lo, dump_rows)])

    return spmm


def _prep_edges(edge_index, edge_weight, e_pad, n):
    e = edge_index.shape[1]
    pad = e_pad - e
    pad_idx = jnp.arange(pad, dtype=jnp.int32) % n
    src = jnp.concatenate([edge_index[1], pad_idx])
    dst = jnp.concatenate([edge_index[0], pad_idx])
    w = jnp.concatenate([edge_weight, jnp.zeros((pad,), jnp.float32)])
    return jnp.stack([src, dst, jax.lax.bitcast_convert_type(w, jnp.int32)])


def _attention(S_e, b0e, b1e, S_ui, b0ui, b1ui, x, Wte, bte, Wtui, btui, W1, b1, W2):
    n = x.shape[0]
    bn = 2000
    nb = n // bn

    def body(se0, se1, su0, su1, x_ref, wte, bter, wtui, btuir, w1, b1r, w2, out_ref):
        xv = x_ref[...]
        z0 = (jnp.concatenate([se0[...], se1[...]], axis=1)
              + jnp.dot(xv, wte[...], preferred_element_type=jnp.float32) + bter[...])
        z1 = (jnp.concatenate([su0[...], su1[...]], axis=1)
              + jnp.dot(xv, wtui[...], preferred_element_type=jnp.float32) + btuir[...])
        t0 = jnp.tanh(jnp.dot(z0, w1[...], preferred_element_type=jnp.float32) + b1r[...])
        t1 = jnp.tanh(jnp.dot(z1, w1[...], preferred_element_type=jnp.float32) + b1r[...])
        s0 = jnp.sum(t0 * w2[...], axis=1, keepdims=True)
        s1 = jnp.sum(t1 * w2[...], axis=1, keepdims=True)
        m = jnp.maximum(s0, s1)
        e0 = jnp.exp(s0 - m)
        e1 = jnp.exp(s1 - m)
        out_ref[...] = (e0 * z0 + e1 * z1) / (e0 + e1)

    return pl.pallas_call(
        body,
        grid=(nb,),
        in_specs=[
            pl.BlockSpec((bn, 64), lambda i, b=b0e // bn: (b + i, 0)),
            pl.BlockSpec((bn, 64), lambda i, b=b1e // bn: (b + i, 0)),
            pl.BlockSpec((bn, 64), lambda i, b=b0ui // bn: (b + i, 0)),
            pl.BlockSpec((bn, 64), lambda i, b=b1ui // bn: (b + i, 0)),
            pl.BlockSpec((bn, FDIM), lambda i: (i, 0)),
            pl.BlockSpec((FDIM, FDIM), lambda i: (0, 0)),
            pl.BlockSpec((1, FDIM), lambda i: (0, 0)),
            pl.BlockSpec((FDIM, FDIM), lambda i: (0, 0)),
            pl.BlockSpec((1, FDIM), lambda i: (0, 0)),
            pl.BlockSpec((FDIM, 32), lambda i: (0, 0)),
            pl.BlockSpec((1, 32), lambda i: (0, 0)),
            pl.BlockSpec((1, 32), lambda i: (0, 0)),
        ],
        out_specs=pl.BlockSpec((bn, FDIM), lambda i: (i, 0)),
        out_shape=jax.ShapeDtypeStruct((n, FDIM), jnp.float32),
    )(S_e, S_e, S_ui, S_ui, x, Wte, bte.reshape(1, FDIM), Wtui,
      btui.reshape(1, FDIM), W1, b1.reshape(1, 32), W2.reshape(1, 32))


def kernel(u_feature, i_feature, u2i_edge_index, u2i_edge_weight,
           u2e_edge_index, u2e_edge_weight, i2e_edge_index, i2e_edge_weight,
           u2e_Wt, u2e_bt, u2e_Wi, u2e_bi,
           i2e_Wt, i2e_bt, i2e_Wi, i2e_bi,
           u2i_Wt, u2i_bt, u2i_Wi, u2i_bi,
           uatt_W1, uatt_b1, uatt_W2,
           iatt_W1, iatt_b1, iatt_W2):
    n_ui = N_U + N_I
    feats = jnp.concatenate([u_feature, i_feature], axis=0)

    g_e = _dense_cell(u_feature, u2e_Wt, u2e_bt, u2e_Wi, u2e_bi)
    g_i = _dense_cell(i_feature, i2e_Wt, i2e_bt, i2e_Wi, i2e_bi)
    g_ui = _dense_cell(feats, u2i_Wt, u2i_bt, u2i_Wi, u2i_bi)

    nch_small = -(-160000 // (NS * EB * 2)) * 2
    nch_big = -(-320000 // (NS * EB * 2)) * 2
    e_small = NS * nch_small * EB
    e_big = NS * nch_big * EB
    ed_e = _prep_edges(u2e_edge_index, u2e_edge_weight, e_small, N_U)
    ed_i = _prep_edges(i2e_edge_index, i2e_edge_weight, e_small, N_I)
    ed_ui = _prep_edges(u2i_edge_index, u2i_edge_weight, e_big, n_ui)

    spmm_small = _make_spmm(N_U, nch_small)
    spmm_big = _make_spmm(n_ui, nch_big)
    s_e = spmm_small(g_e, ed_e)
    s_i = spmm_small(g_i, ed_i)
    s_ui = spmm_big(g_ui, ed_ui)

    u_out = _attention(s_e, 0, N_U, s_ui, 0, n_ui, u_feature,
                       u2e_Wt, u2e_bt, u2i_Wt, u2i_bt, uatt_W1, uatt_b1, uatt_W2)
    i_out = _attention(s_i, 0, N_I, s_ui, N_U, n_ui + N_U, i_feature,
                       i2e_Wt, i2e_bt, u2i_Wt, u2i_bt, iatt_W1, iatt_b1, iatt_W2)
    return (u_out, i_out)


# speedup vs baseline: 14.7342x; 1.2554x over previous
import functools

import jax
import jax.numpy as jnp
from jax import lax
from jax.experimental import pallas as pl
from jax.experimental.pallas import tpu as pltpu
from jax.experimental.pallas import tpu_sc as plsc

N_U = 10000
N_I = 10000
FDIM = 128
NS = 16
NC = 2
EB = 128


def _dense_cell(x, Wt, bt, Wi, bi):
    n = x.shape[0]
    bn = 2000
    nb = n // bn

    def body(x_ref, wt_ref, bt_ref, wi_ref, bi_ref, g_ref):
        h = pl.program_id(1)
        xv = x_ref[...]
        h1 = jnp.dot(xv, wt_ref[...], preferred_element_type=jnp.float32) + bt_ref[...]
        h12 = h1 + jnp.dot(xv * xv, wi_ref[...], preferred_element_type=jnp.float32) + bi_ref[...]
        g_ref[...] = jnp.where(h == 0, h12[:, :64], h12[:, 64:])

    return pl.pallas_call(
        body,
        grid=(nb, 2),
        in_specs=[
            pl.BlockSpec((bn, FDIM), lambda i, h: (i, 0)),
            pl.BlockSpec((FDIM, FDIM), lambda i, h: (0, 0)),
            pl.BlockSpec((1, FDIM), lambda i, h: (0, 0)),
            pl.BlockSpec((FDIM, FDIM), lambda i, h: (0, 0)),
            pl.BlockSpec((1, FDIM), lambda i, h: (0, 0)),
        ],
        out_specs=pl.BlockSpec((bn, 64), lambda i, h: (h * nb + i, 0)),
        out_shape=jax.ShapeDtypeStruct((2 * n, 64), jnp.float32),
    )(x, Wt, bt.reshape(1, FDIM), Wi, bi.reshape(1, FDIM))


@functools.lru_cache(maxsize=None)
def _make_spmm(n, n_chunks):
    e_per_tile = n_chunks * EB
    dump_rows = 200
    dump_chunks = n // dump_rows
    dump_iters = (dump_chunks + NS - 1) // NS
    mesh = plsc.VectorSubcoreMesh(
        core_axis_name="c", subcore_axis_name="s", num_cores=NC, num_subcores=NS)

    @functools.partial(
        pl.kernel,
        out_type=jax.ShapeDtypeStruct((2 * n, 64), jnp.float32),
        mesh=mesh,
        scratch_types=(
            [pltpu.VMEM((3, EB), jnp.int32)] * 4
            + [pltpu.VMEM((EB,), jnp.int32)] * 4
            + [pltpu.VMEM((EB, 64), jnp.float32)] * 4
            + [pltpu.VMEM((dump_rows, 64), jnp.float32)]
            + [pltpu.VMEM_SHARED((n, 64), jnp.float32)]
            + [pltpu.SemaphoreType.DMA] * 12
        ),
        compiler_params=pltpu.CompilerParams(
            use_tc_tiling_on_sc=False, needs_layout_passes=False),
    )
    def spmm(g_hbm, edata_hbm, out_hbm, *scr):
        ibuf = scr[0:4]
        dbuf = scr[4:8]
        rows = scr[8:12]
        zbuf = scr[12]
        acc = scr[13]
        sem_i = scr[14:18]
        sem_g = scr[18:22]
        sem_c = scr[22:26]
        c = lax.axis_index("c")
        s = lax.axis_index("s")
        cn = c * n
        base = s * e_per_tile

        def idx_desc(t, b):
            tc = jnp.minimum(t, n_chunks - 1)
            return pltpu.make_async_copy(
                edata_hbm.at[:, pl.ds(base + tc * EB, EB)], ibuf[b], sem_i[b])

        def gather_desc(t, b):
            return pltpu.make_async_copy(
                g_hbm.at[ibuf[b].at[0]], rows[b], sem_g[b])

        def scatter_start(b):
            pltpu.async_copy(rows[b], acc.at[dbuf[b]], sem_c[b], add=True)

        def scatter_wait(b):
            pltpu.make_async_copy(rows[b], acc.at[dbuf[b]], sem_c[b]).wait()

        def adjust(b):
            for k in range(EB // 16):
                sl = pl.ds(16 * k, 16)
                ibuf[b][0, sl] = ibuf[b][0, sl] + cn

        def scale(b):
            @plsc.parallel_loop(0, EB // 16, unroll=2)
            def _(g):
                wvec = plsc.bitcast(ibuf[b][2, pl.ds(16 * g, 16)], jnp.float32)
                for jj in range(16):
                    wj = wvec[jj]
                    j = 16 * g + jj
                    for k in range(4):
                        sl = pl.ds(16 * k, 16)
                        rows[b][j, sl] = rows[b][j, sl] * wj

        def zb(j, carry):
            for k in range(4):
                zbuf[j, pl.ds(16 * k, 16)] = jnp.zeros((16,), jnp.float32)
            return carry
        lax.fori_loop(0, dump_rows, zb, 0)
        for t in range(dump_iters):
            q = s + NS * t
            @pl.when(q < dump_chunks)
            def _():
                pltpu.sync_copy(zbuf, acc.at[pl.ds(q * dump_rows, dump_rows)])
        plsc.subcore_barrier()

        for b in range(4):
            idx_desc(b, b).start()
        for b in range(2):
            idx_desc(b, b).wait()
            adjust(b)
            gather_desc(b, b).start()

        def step(t, b):
            bp2 = (b + 2) % 4
            idx_desc(t + 2, bp2).wait()
            adjust(bp2)
            @pl.when(t > 1)
            def _():
                scatter_wait(bp2)
            gather_desc(t + 2, bp2).start()
            gather_desc(t, b).wait()
            scale(b)
            for k in range(EB // 16):
                sl = pl.ds(16 * k, 16)
                dbuf[b][sl] = ibuf[b][1, sl]
            scatter_start(b)
            idx_desc(t + 4, b).start()

        def body(tt, carry):
            for b in range(4):
                step(4 * tt + b, b)
            return carry
        lax.fori_loop(0, n_chunks // 4, body, 0)

        nc = n_chunks
        idx_desc(nc + 2, (nc + 2) % 4).wait()
        idx_desc(nc + 3, (nc + 3) % 4).wait()
        gather_desc(nc, nc % 4).wait()
        gather_desc(nc + 1, (nc + 1) % 4).wait()
        scatter_wait((nc - 2) % 4)
        scatter_wait((nc - 1) % 4)
        plsc.subcore_barrier()

        for t in range(dump_iters):
            q = s + NS * t
            @pl.when(q < dump_chunks)
            def _():
                lo = q * dump_rows
                pltpu.sync_copy(acc.at[pl.ds(lo, dump_rows)],
                                out_hbm.at[pl.ds(cn + lo, dump_rows)])

    return spmm


def _prep_edges(edge_index, edge_weight, e_pad, n):
    e = edge_index.shape[1]
    pad = e_pad - e
    pad_idx = jnp.arange(pad, dtype=jnp.int32) % n
    src = jnp.concatenate([edge_index[1], pad_idx])
    dst = jnp.concatenate([edge_index[0], pad_idx])
    w = jnp.concatenate([edge_weight, jnp.zeros((pad,), jnp.float32)])
    return jnp.stack([src, dst, jax.lax.bitcast_convert_type(w, jnp.int32)])


def _attention(S_e, b0e, b1e, S_ui, b0ui, b1ui, x, Wte, bte, Wtui, btui, W1, b1, W2):
    n = x.shape[0]
    bn = 2000
    nb = n // bn

    def body(se0, se1, su0, su1, x_ref, wte, bter, wtui, btuir, w1, b1r, w2, out_ref):
        xv = x_ref[...]
        z0 = (jnp.concatenate([se0[...], se1[...]], axis=1)
              + jnp.dot(xv, wte[...], preferred_element_type=jnp.float32) + bter[...])
        z1 = (jnp.concatenate([su0[...], su1[...]], axis=1)
              + jnp.dot(xv, wtui[...], preferred_element_type=jnp.float32) + btuir[...])
        t0 = jnp.tanh(jnp.dot(z0, w1[...], preferred_element_type=jnp.float32) + b1r[...])
        t1 = jnp.tanh(jnp.dot(z1, w1[...], preferred_element_type=jnp.float32) + b1r[...])
        s0 = jnp.sum(t0 * w2[...], axis=1, keepdims=True)
        s1 = jnp.sum(t1 * w2[...], axis=1, keepdims=True)
        m = jnp.maximum(s0, s1)
        e0 = jnp.exp(s0 - m)
        e1 = jnp.exp(s1 - m)
        out_ref[...] = (e0 * z0 + e1 * z1) / (e0 + e1)

    return pl.pallas_call(
        body,
        grid=(nb,),
        in_specs=[
            pl.BlockSpec((bn, 64), lambda i, b=b0e // bn: (b + i, 0)),
            pl.BlockSpec((bn, 64), lambda i, b=b1e // bn: (b + i, 0)),
            pl.BlockSpec((bn, 64), lambda i, b=b0ui // bn: (b + i, 0)),
            pl.BlockSpec((bn, 64), lambda i, b=b1ui // bn: (b + i, 0)),
            pl.BlockSpec((bn, FDIM), lambda i: (i, 0)),
            pl.BlockSpec((FDIM, FDIM), lambda i: (0, 0)),
            pl.BlockSpec((1, FDIM), lambda i: (0, 0)),
            pl.BlockSpec((FDIM, FDIM), lambda i: (0, 0)),
            pl.BlockSpec((1, FDIM), lambda i: (0, 0)),
            pl.BlockSpec((FDIM, 32), lambda i: (0, 0)),
            pl.BlockSpec((1, 32), lambda i: (0, 0)),
            pl.BlockSpec((1, 32), lambda i: (0, 0)),
        ],
        out_specs=pl.BlockSpec((bn, FDIM), lambda i: (i, 0)),
        out_shape=jax.ShapeDtypeStruct((n, FDIM), jnp.float32),
    )(S_e, S_e, S_ui, S_ui, x, Wte, bte.reshape(1, FDIM), Wtui,
      btui.reshape(1, FDIM), W1, b1.reshape(1, 32), W2.reshape(1, 32))


def kernel(u_feature, i_feature, u2i_edge_index, u2i_edge_weight,
           u2e_edge_index, u2e_edge_weight, i2e_edge_index, i2e_edge_weight,
           u2e_Wt, u2e_bt, u2e_Wi, u2e_bi,
           i2e_Wt, i2e_bt, i2e_Wi, i2e_bi,
           u2i_Wt, u2i_bt, u2i_Wi, u2i_bi,
           uatt_W1, uatt_b1, uatt_W2,
           iatt_W1, iatt_b1, iatt_W2):
    n_ui = N_U + N_I
    feats = jnp.concatenate([u_feature, i_feature], axis=0)

    g_e = _dense_cell(u_feature, u2e_Wt, u2e_bt, u2e_Wi, u2e_bi)
    g_i = _dense_cell(i_feature, i2e_Wt, i2e_bt, i2e_Wi, i2e_bi)
    g_ui = _dense_cell(feats, u2i_Wt, u2i_bt, u2i_Wi, u2i_bi)

    nch_small = -(-160000 // (NS * EB * 4)) * 4
    nch_big = -(-320000 // (NS * EB * 4)) * 4
    e_small = NS * nch_small * EB
    e_big = NS * nch_big * EB
    ed_e = _prep_edges(u2e_edge_index, u2e_edge_weight, e_small, N_U)
    ed_i = _prep_edges(i2e_edge_index, i2e_edge_weight, e_small, N_I)
    ed_ui = _prep_edges(u2i_edge_index, u2i_edge_weight, e_big, n_ui)

    spmm_small = _make_spmm(N_U, nch_small)
    spmm_big = _make_spmm(n_ui, nch_big)
    s_e = spmm_small(g_e, ed_e)
    s_i = spmm_small(g_i, ed_i)
    s_ui = spmm_big(g_ui, ed_ui)

    u_out = _attention(s_e, 0, N_U, s_ui, 0, n_ui, u_feature,
                       u2e_Wt, u2e_bt, u2i_Wt, u2i_bt, uatt_W1, uatt_b1, uatt_W2)
    i_out = _attention(s_i, 0, N_I, s_ui, N_U, n_ui + N_U, i_feature,
                       i2e_Wt, i2e_bt, u2i_Wt, u2i_bt, iatt_W1, iatt_b1, iatt_W2)
    return (u_out, i_out)
